# baseline (speedup 1.0000x reference)
import functools

import jax
import jax.numpy as jnp
from jax import lax
from jax.experimental import pallas as pl
from jax.experimental.pallas import tpu as pltpu
from jax.experimental.pallas import tpu_sc as plsc

V = 100000
D = 128
B = 4096 * 200
NC = 2
NS = 16
NW = NC * NS
BPW = B // NW
CH = 64
NBUF = 4
LA = 2
NCH = BPW // CH
EPR = 128 // 4
TROWS = 3200
TPW = TROWS // NS

assert NCH % NBUF == 0 and CH % 16 == 0 and LA < NBUF
assert V <= TROWS * EPR

_mesh = plsc.VectorSubcoreMesh(core_axis_name="c", subcore_axis_name="s")


@functools.partial(
    pl.kernel,
    mesh=_mesh,
    out_type=jax.ShapeDtypeStruct((B, D), jnp.float32),
    scratch_types=(
        [pltpu.VMEM((BPW,), jnp.int32)]
        + [pltpu.VMEM((CH,), jnp.int32) for _ in range(NBUF)]
        + [pltpu.VMEM((CH, D), jnp.int32) for _ in range(NBUF)]
        + [pltpu.VMEM((CH, D), jnp.float32) for _ in range(NBUF)]
        + [pltpu.VMEM_SHARED((TROWS, D), jnp.int32)]
        + [pltpu.SemaphoreType.DMA for _ in range(2 * NBUF)]
    ),
)
def _sc_gather(tok_hbm, packed_hbm, out_hbm, idx_v, *rest):
    rvbufs = rest[:NBUF]
    pbufs = rest[NBUF : 2 * NBUF]
    fbufs = rest[2 * NBUF : 3 * NBUF]
    spm_tab = rest[3 * NBUF]
    sem_g = rest[3 * NBUF + 1 : 4 * NBUF + 1]
    sem_w = rest[4 * NBUF + 1 :]
    sid = lax.axis_index("s")
    wid = sid * NC + lax.axis_index("c")
    base = wid * BPW

    pltpu.sync_copy(tok_hbm.at[pl.ds(base, BPW)], idx_v)
    pltpu.sync_copy(
        packed_hbm.at[pl.ds(sid * TPW, TPW)], spm_tab.at[pl.ds(sid * TPW, TPW)]
    )

    plsc.subcore_barrier()

    def start_gather(c, b):
        for i in range(CH // 16):
            rvbufs[b][pl.ds(i * 16, 16)] = lax.shift_right_logical(
                idx_v[pl.ds(c * CH + i * 16, 16)], jnp.int32(5)
            )
        pltpu.async_copy(spm_tab.at[rvbufs[b]], pbufs[b], sem_g[b])

    def wait_gather(c, b):
        pltpu.make_async_copy(
            spm_tab.at[rvbufs[b]], pbufs[b], sem_g[b]
        ).wait()

    def start_write(c, b):
        pltpu.async_copy(fbufs[b], out_hbm.at[pl.ds(base + c * CH, CH)], sem_w[b])

    def wait_write(c, b):
        pltpu.make_async_copy(
            fbufs[b], out_hbm.at[pl.ds(base + c * CH, CH)], sem_w[b]
        ).wait()

    sv = [lax.iota(jnp.int32, 16), lax.iota(jnp.int32, 16) + jnp.int32(16)]
    one = jnp.int32(1)
    dnums = lax.GatherDimensionNumbers(
        offset_dims=(), collapsed_slice_dims=(0,), start_index_map=(0,)
    )

    def expand(c, b):
        pb = pbufs[b]
        fb = fbufs[b]

        def tokens(ti, carry):
            t0 = ti * 16
            vt = idx_v[pl.ds(c * CH + t0, 16)]
            for dt in range(16):
                t = t0 + dt
                e = lax.bitwise_and(vt[dt], jnp.int32(31))
                wstart = lax.shift_left(
                    lax.shift_right_logical(e, jnp.int32(2)), jnp.int32(4)
                )
                win = pb[t, pl.ds(wstart, 16)]
                p0 = lax.shift_left(
                    lax.bitwise_and(e, jnp.int32(3)), jnp.int32(2)
                )
                for g in range(8):
                    pidx = jnp.full((16, 1), p0 + jnp.int32(g // 2), jnp.int32)
                    wv = lax.gather(
                        win,
                        pidx,
                        dnums,
                        (1,),
                        mode=lax.GatherScatterMode.PROMISE_IN_BOUNDS,
                    )
                    bits = lax.bitwise_and(
                        lax.shift_right_logical(wv, sv[g % 2]), one
                    )
                    fb[t, pl.ds(g * 16, 16)] = lax.convert_element_type(
                        bits, jnp.float32
                    )
            return carry

        lax.fori_loop(0, CH // 16, tokens, 0)

    for b in range(LA):
        start_gather(b, b)

    def group(gi, carry):
        c0 = gi * NBUF
        for b in range(NBUF):
            c = c0 + b
            nxt = c + LA
            sb = (b + LA) % NBUF

            @pl.when(nxt < NCH)
            def _():
                start_gather(nxt, sb)

            wait_gather(c, b)

            @pl.when(c >= NBUF)
            def _():
                wait_write(c - NBUF, b)

            expand(c, b)
            start_write(c, b)

        return carry

    lax.fori_loop(0, NCH // NBUF, group, 0)

    for b in range(NBUF):
        wait_write(NCH - NBUF + b, b)


def kernel(tokens, codes):
    idx = tokens.reshape(-1).astype(jnp.int32)
    ci = codes.astype(jnp.uint32).reshape(V, 4, 32)
    words = (ci << jnp.arange(32, dtype=jnp.uint32)).sum(
        axis=-1, dtype=jnp.uint32
    )
    words = jax.lax.bitcast_convert_type(words, jnp.int32).reshape(
        V // EPR, D
    )
    packed = jnp.concatenate(
        [words, jnp.zeros((TROWS - V // EPR, D), jnp.int32)], axis=0
    )
    out = _sc_gather(idx, packed)
    return out.reshape(tokens.shape + (D,))


# speedup vs baseline: 1.5167x; 1.5167x over previous
import functools

import jax
import jax.numpy as jnp
from jax import lax
from jax.experimental import pallas as pl
from jax.experimental.pallas import tpu as pltpu
from jax.experimental.pallas import tpu_sc as plsc

V = 100000
D = 128
B = 4096 * 200
NC = 2
NS = 16
NW = NC * NS
BPW = B // NW
CH = 128
NBUF = 4
NCH = BPW // CH

assert NCH % NBUF == 0

_mesh = plsc.VectorSubcoreMesh(core_axis_name="c", subcore_axis_name="s")


@functools.partial(
    pl.kernel,
    mesh=_mesh,
    out_type=jax.ShapeDtypeStruct((B, D), jnp.float32),
    scratch_types=(
        [pltpu.VMEM((BPW,), jnp.int32)]
        + [pltpu.VMEM((CH, D), jnp.float32) for _ in range(NBUF)]
        + [pltpu.SemaphoreType.DMA for _ in range(NBUF)]
    ),
)
def _sc_gather(tok_hbm, codes_hbm, out_hbm, idx_v, *bufs_sems):
    bufs = bufs_sems[:NBUF]
    sems = bufs_sems[NBUF:]
    wid = lax.axis_index("s") * NC + lax.axis_index("c")
    base = wid * BPW

    pltpu.sync_copy(tok_hbm.at[pl.ds(base, BPW)], idx_v)

    for b in range(NBUF):
        pltpu.async_copy(
            codes_hbm.at[idx_v.at[pl.ds(b * CH, CH)]], bufs[b], sems[b]
        )

    def group(gi, carry):
        c0 = gi * NBUF
        for b in range(NBUF):
            c = c0 + b
            pltpu.make_async_copy(
                codes_hbm.at[idx_v.at[pl.ds(c * CH, CH)]], bufs[b], sems[b]
            ).wait()
            pltpu.sync_copy(bufs[b], out_hbm.at[pl.ds(base + c * CH, CH)])
            nxt = c + NBUF

            @pl.when(nxt < NCH)
            def _():
                pltpu.async_copy(
                    codes_hbm.at[idx_v.at[pl.ds(nxt * CH, CH)]], bufs[b], sems[b]
                )

        return carry

    lax.fori_loop(0, NCH // NBUF, group, 0)


def kernel(tokens, codes):
    idx = tokens.reshape(-1).astype(jnp.int32)
    out = _sc_gather(idx, codes)
    return out.reshape(tokens.shape + (D,))
